# initial kernel scaffold (unmeasured)
import functools

import jax
import jax.numpy as jnp
from jax import lax
from jax.experimental import pallas as pl
from jax.experimental.pallas import tpu as pltpu

N_DEV = 4
SQ = 512
D = 1024
SKV = 2048
HQ_PER = 8
DH = 128
SCALE = 0.08838834764831843


def kernel(x, Wq, Wo, K_ext, V_ext):
    x2 = x.reshape(SQ, D)

    def body(x_ref, wq_ref, wo_ref, k_hbm, v_hbm, out_ref,
             xs_ref, acc_ref, kb_ref, vb_ref, comm_ref,
             ag_send, ag_recv, rs_send, rs_recv, kv_sems):
        my = lax.axis_index("i")
        left = lax.rem(my + N_DEV - 1, N_DEV)
        right = lax.rem(my + 1, N_DEV)

        barrier = pltpu.get_barrier_semaphore()
        for nbr in (left, right):
            pl.semaphore_signal(barrier, inc=1, device_id=(nbr,),
                                device_id_type=pl.DeviceIdType.MESH)
        pl.semaphore_wait(barrier, 2)

        xs_ref[my] = x_ref[...]

        for h in range(N_DEV - 1):
            src_o = lax.rem(my + N_DEV - h, N_DEV)
            rdma = pltpu.make_async_remote_copy(
                src_ref=xs_ref.at[src_o],
                dst_ref=xs_ref.at[src_o],
                send_sem=ag_send.at[h],
                recv_sem=ag_recv.at[h],
                device_id=(right,),
                device_id_type=pl.DeviceIdType.MESH,
            )
            rdma.start()
            rdma.wait()

        def compute_partial(b):
            kcp = pltpu.make_async_copy(
                k_hbm.at[b, :, pl.ds(my * HQ_PER, HQ_PER), :],
                kb_ref, kv_sems.at[0])
            vcp = pltpu.make_async_copy(
                v_hbm.at[b, :, pl.ds(my * HQ_PER, HQ_PER), :],
                vb_ref, kv_sems.at[1])
            kcp.start()
            vcp.start()
            q = jnp.dot(xs_ref[b], wq_ref[...],
                        preferred_element_type=jnp.float32)
            kcp.wait()
            vcp.wait()
            ohs = []
            for h in range(HQ_PER):
                qh = q[:, h * DH:(h + 1) * DH]
                kh = kb_ref[:, h, :]
                s = lax.dot_general(
                    qh, kh, (((1,), (1,)), ((), ())),
                    preferred_element_type=jnp.float32) * SCALE
                m = jnp.max(s, axis=1, keepdims=True)
                p = jnp.exp(s - m)
                l = jnp.sum(p, axis=1, keepdims=True)
                oh = jnp.dot(p, vb_ref[:, h, :],
                             preferred_element_type=jnp.float32) / l
                ohs.append(oh)
            o = jnp.concatenate(ohs, axis=1)
            acc_ref[b] = jnp.dot(o, wo_ref[...],
                                 preferred_element_type=jnp.float32)

        for b in range(N_DEV):
            compute_partial(b)

        for s in range(N_DEV - 1):
            a_s = lax.rem(my + 2 * N_DEV - 1 - s, N_DEV)
            rdma = pltpu.make_async_remote_copy(
                src_ref=acc_ref.at[a_s],
                dst_ref=comm_ref.at[s],
                send_sem=rs_send.at[s],
                recv_sem=rs_recv.at[s],
                device_id=(right,),
                device_id_type=pl.DeviceIdType.MESH,
            )
            rdma.start()
            rdma.wait()
            a_n = lax.rem(my + 2 * N_DEV - 2 - s, N_DEV)
            acc_ref[a_n] = acc_ref[a_n] + comm_ref[s]

        out_ref[0] = acc_ref[my]

        @functools.partial(pl.run_scoped,
                           exit_sem=pltpu.SemaphoreType.REGULAR)
        def _(exit_sem):
            for nbr in (left, right):
                pl.semaphore_signal(exit_sem, inc=1, device_id=(nbr,),
                                    device_id_type=pl.DeviceIdType.MESH)
            pl.semaphore_wait(exit_sem, 2)

    return pl.pallas_call(
        body,
        out_shape=jax.ShapeDtypeStruct((1, SQ, D), jnp.float32),
        in_specs=[
            pl.BlockSpec(memory_space=pltpu.VMEM),
            pl.BlockSpec(memory_space=pltpu.VMEM),
            pl.BlockSpec(memory_space=pltpu.VMEM),
            pl.BlockSpec(memory_space=pltpu.ANY),
            pl.BlockSpec(memory_space=pltpu.ANY),
        ],
        out_specs=pl.BlockSpec(memory_space=pltpu.VMEM),
        scratch_shapes=[
            pltpu.VMEM((N_DEV, SQ, D), jnp.float32),
            pltpu.VMEM((N_DEV, SQ, D), jnp.float32),
            pltpu.VMEM((SKV, HQ_PER, DH), jnp.float32),
            pltpu.VMEM((SKV, HQ_PER, DH), jnp.float32),
            pltpu.VMEM((N_DEV - 1, SQ, D), jnp.float32),
            pltpu.SemaphoreType.DMA((N_DEV - 1,)),
            pltpu.SemaphoreType.DMA((N_DEV - 1,)),
            pltpu.SemaphoreType.DMA((N_DEV - 1,)),
            pltpu.SemaphoreType.DMA((N_DEV - 1,)),
            pltpu.SemaphoreType.DMA((2,)),
        ],
        compiler_params=pltpu.CompilerParams(collective_id=0),
    )(x2, Wq, Wo, K_ext, V_ext)


# baseline (device time: 289288 ns/iter reference)
import functools

import jax
import jax.numpy as jnp
from jax import lax
from jax.experimental import pallas as pl
from jax.experimental.pallas import tpu as pltpu

N_DEV = 4
SQ = 512
D = 1024
SKV = 2048
HQ_PER = 8
DH = 128
SCALE = 0.08838834764831843


def kernel(x, Wq, Wo, K_ext, V_ext):
    x2 = x.reshape(SQ, D)

    def body(x_ref, wq_ref, wo_ref, k_hbm, v_hbm, out_ref,
             xs_ref, acc_ref, kb_ref, vb_ref, q_ref, o_ref, comm_ref,
             ag_send, ag_recv, rs_send, rs_recv, kv_sems):
        my = lax.axis_index("i")
        left = lax.rem(my + N_DEV - 1, N_DEV)
        right = lax.rem(my + 1, N_DEV)

        barrier = pltpu.get_barrier_semaphore()
        for nbr in (left, right):
            pl.semaphore_signal(barrier, inc=1, device_id=(nbr,),
                                device_id_type=pl.DeviceIdType.MESH)
        pl.semaphore_wait(barrier, 2)

        xs_ref[my] = x_ref[...]

        for h in range(N_DEV - 1):
            src_o = lax.rem(my + N_DEV - h, N_DEV)
            rdma = pltpu.make_async_remote_copy(
                src_ref=xs_ref.at[src_o],
                dst_ref=xs_ref.at[src_o],
                send_sem=ag_send.at[h],
                recv_sem=ag_recv.at[h],
                device_id=(right,),
                device_id_type=pl.DeviceIdType.MESH,
            )
            rdma.start()
            rdma.wait()

        def batch_body(b, _):
            for hh in range(HQ_PER):
                pltpu.make_async_copy(
                    k_hbm.at[b, :, my * HQ_PER + hh, :],
                    kb_ref.at[hh], kv_sems.at[0]).start()
                pltpu.make_async_copy(
                    v_hbm.at[b, :, my * HQ_PER + hh, :],
                    vb_ref.at[hh], kv_sems.at[1]).start()
            q_ref[...] = jnp.dot(xs_ref[b], wq_ref[...],
                                 preferred_element_type=jnp.float32)
            for hh in range(HQ_PER):
                pltpu.make_async_copy(
                    k_hbm.at[b, :, my * HQ_PER + hh, :],
                    kb_ref.at[hh], kv_sems.at[0]).wait()
                pltpu.make_async_copy(
                    v_hbm.at[b, :, my * HQ_PER + hh, :],
                    vb_ref.at[hh], kv_sems.at[1]).wait()

            def head_body(h, _):
                qh = q_ref[:, pl.ds(h * DH, DH)]
                s = lax.dot_general(
                    qh, kb_ref[h], (((1,), (1,)), ((), ())),
                    preferred_element_type=jnp.float32) * SCALE
                m = jnp.max(s, axis=1, keepdims=True)
                p = jnp.exp(s - m)
                l = jnp.sum(p, axis=1, keepdims=True)
                oh = jnp.dot(p, vb_ref[h],
                             preferred_element_type=jnp.float32) / l
                o_ref[:, pl.ds(h * DH, DH)] = oh
                return 0

            lax.fori_loop(0, HQ_PER, head_body, 0)
            acc_ref[b] = jnp.dot(o_ref[...], wo_ref[...],
                                 preferred_element_type=jnp.float32)
            return 0

        lax.fori_loop(0, N_DEV, batch_body, 0)

        for s in range(N_DEV - 1):
            a_s = lax.rem(my + 2 * N_DEV - 1 - s, N_DEV)
            rdma = pltpu.make_async_remote_copy(
                src_ref=acc_ref.at[a_s],
                dst_ref=comm_ref.at[s],
                send_sem=rs_send.at[s],
                recv_sem=rs_recv.at[s],
                device_id=(right,),
                device_id_type=pl.DeviceIdType.MESH,
            )
            rdma.start()
            rdma.wait()
            a_n = lax.rem(my + 2 * N_DEV - 2 - s, N_DEV)
            acc_ref[a_n] = acc_ref[a_n] + comm_ref[s]

        out_ref[0] = acc_ref[my]

        @functools.partial(pl.run_scoped,
                           exit_sem=pltpu.SemaphoreType.REGULAR)
        def _(exit_sem):
            for nbr in (left, right):
                pl.semaphore_signal(exit_sem, inc=1, device_id=(nbr,),
                                    device_id_type=pl.DeviceIdType.MESH)
            pl.semaphore_wait(exit_sem, 2)

    return pl.pallas_call(
        body,
        out_shape=jax.ShapeDtypeStruct((1, SQ, D), jnp.float32),
        in_specs=[
            pl.BlockSpec(memory_space=pltpu.MemorySpace.VMEM),
            pl.BlockSpec(memory_space=pltpu.MemorySpace.VMEM),
            pl.BlockSpec(memory_space=pltpu.MemorySpace.VMEM),
            pl.BlockSpec(memory_space=pl.ANY),
            pl.BlockSpec(memory_space=pl.ANY),
        ],
        out_specs=pl.BlockSpec(memory_space=pltpu.MemorySpace.VMEM),
        scratch_shapes=[
            pltpu.VMEM((N_DEV, SQ, D), jnp.float32),
            pltpu.VMEM((N_DEV, SQ, D), jnp.float32),
            pltpu.VMEM((HQ_PER, SKV, DH), jnp.float32),
            pltpu.VMEM((HQ_PER, SKV, DH), jnp.float32),
            pltpu.VMEM((SQ, D), jnp.float32),
            pltpu.VMEM((SQ, D), jnp.float32),
            pltpu.VMEM((N_DEV - 1, SQ, D), jnp.float32),
            pltpu.SemaphoreType.DMA((N_DEV - 1,)),
            pltpu.SemaphoreType.DMA((N_DEV - 1,)),
            pltpu.SemaphoreType.DMA((N_DEV - 1,)),
            pltpu.SemaphoreType.DMA((N_DEV - 1,)),
            pltpu.SemaphoreType.DMA((2,)),
        ],
        compiler_params=pltpu.CompilerParams(
            collective_id=0,
            vmem_limit_bytes=100 * 1024 * 1024,
        ),
    )(x2, Wq, Wo, K_ext, V_ext)


# device time: 165750 ns/iter; 1.7453x vs baseline; 1.7453x over previous
import functools

import jax
import jax.numpy as jnp
from jax import lax
from jax.experimental import pallas as pl
from jax.experimental.pallas import tpu as pltpu

N_DEV = 4
SQ = 512
D = 1024
SKV = 2048
HQ_PER = 8
DH = 128
SCALE = 0.08838834764831843


def kernel(x, Wq, Wo, K_ext, V_ext):
    x2 = x.reshape(SQ, D)

    def body(x_ref, wq_ref, wo_ref, k_hbm, v_hbm, out_ref,
             xs_ref, acc_ref, kb_ref, vb_ref, q_ref, o_ref, comm_ref,
             ag_send, ag_recv, rs_send, rs_recv, ksems, vsems):
        my = lax.axis_index("i")
        left = lax.rem(my + N_DEV - 1, N_DEV)
        right = lax.rem(my + 1, N_DEV)

        def b_of(t):
            return lax.rem(my + N_DEV - t, N_DEV)

        def kv_descs(b, h, slot):
            k = pltpu.make_async_copy(
                k_hbm.at[b, :, my * HQ_PER + h, :],
                kb_ref.at[slot], ksems.at[slot])
            v = pltpu.make_async_copy(
                v_hbm.at[b, :, my * HQ_PER + h, :],
                vb_ref.at[slot], vsems.at[slot])
            return k, v

        def ag_rdma(h):
            return pltpu.make_async_remote_copy(
                src_ref=xs_ref.at[b_of(h)],
                dst_ref=xs_ref.at[b_of(h)],
                send_sem=ag_send.at[h],
                recv_sem=ag_recv.at[h],
                device_id=(right,),
                device_id_type=pl.DeviceIdType.MESH,
            )

        def rs_rdma(s):
            return pltpu.make_async_remote_copy(
                src_ref=acc_ref.at[b_of(s + 1)],
                dst_ref=comm_ref.at[s],
                send_sem=rs_send.at[s],
                recv_sem=rs_recv.at[s],
                device_id=(right,),
                device_id_type=pl.DeviceIdType.MESH,
            )

        def compute_partial(t):
            b = b_of(t)
            q_ref[...] = jnp.dot(xs_ref[b], wq_ref[...],
                                 preferred_element_type=jnp.float32)

            def head_body(h, _):
                slot = lax.rem(h, 2)
                nk, nv = kv_descs(
                    jnp.where(h < HQ_PER - 1, b, b_of(t + 1)),
                    lax.rem(h + 1, HQ_PER),
                    lax.rem(h + 1, 2),
                )
                if t == N_DEV - 1:
                    @pl.when(h < HQ_PER - 1)
                    def _():
                        nk.start()
                        nv.start()
                else:
                    nk.start()
                    nv.start()
                ck, cv = kv_descs(b, h, slot)
                ck.wait()
                cv.wait()

                qh = q_ref[:, pl.ds(h * DH, DH)]
                s = lax.dot_general(
                    qh, kb_ref[slot], (((1,), (1,)), ((), ())),
                    preferred_element_type=jnp.float32) * SCALE
                m = jnp.max(s, axis=1, keepdims=True)
                p = jnp.exp(s - m)
                l = jnp.sum(p, axis=1, keepdims=True)
                oh = jnp.dot(p, vb_ref[slot],
                             preferred_element_type=jnp.float32) / l
                o_ref[:, pl.ds(h * DH, DH)] = oh
                return 0

            lax.fori_loop(0, HQ_PER, head_body, 0)
            acc_ref[b] = jnp.dot(o_ref[...], wo_ref[...],
                                 preferred_element_type=jnp.float32)

        k0, v0 = kv_descs(b_of(0), 0, 0)
        k0.start()
        v0.start()

        barrier = pltpu.get_barrier_semaphore()
        for nbr in (left, right):
            pl.semaphore_signal(barrier, inc=1, device_id=(nbr,),
                                device_id_type=pl.DeviceIdType.MESH)
        pl.semaphore_wait(barrier, 2)

        xs_ref[my] = x_ref[...]

        ag_rdma(0).start()
        compute_partial(0)

        ag_rdma(0).wait_recv()
        ag_rdma(1).start()
        compute_partial(1)
        rs_rdma(0).start()

        ag_rdma(1).wait_recv()
        ag_rdma(2).start()
        compute_partial(2)
        rs_rdma(0).wait_recv()
        acc_ref[b_of(2)] = acc_ref[b_of(2)] + comm_ref[0]
        rs_rdma(1).start()

        ag_rdma(2).wait_recv()
        compute_partial(3)
        rs_rdma(1).wait_recv()
        acc_ref[b_of(3)] = acc_ref[b_of(3)] + comm_ref[1]
        rs_rdma(2).start()

        rs_rdma(2).wait_recv()
        out_ref[0] = acc_ref[my] + comm_ref[2]

        for h in range(N_DEV - 1):
            ag_rdma(h).wait_send()
            rs_rdma(h).wait_send()

        @functools.partial(pl.run_scoped,
                           exit_sem=pltpu.SemaphoreType.REGULAR)
        def _(exit_sem):
            for nbr in (left, right):
                pl.semaphore_signal(exit_sem, inc=1, device_id=(nbr,),
                                    device_id_type=pl.DeviceIdType.MESH)
            pl.semaphore_wait(exit_sem, 2)

    return pl.pallas_call(
        body,
        out_shape=jax.ShapeDtypeStruct((1, SQ, D), jnp.float32),
        in_specs=[
            pl.BlockSpec(memory_space=pltpu.MemorySpace.VMEM),
            pl.BlockSpec(memory_space=pltpu.MemorySpace.VMEM),
            pl.BlockSpec(memory_space=pltpu.MemorySpace.VMEM),
            pl.BlockSpec(memory_space=pl.ANY),
            pl.BlockSpec(memory_space=pl.ANY),
        ],
        out_specs=pl.BlockSpec(memory_space=pltpu.MemorySpace.VMEM),
        scratch_shapes=[
            pltpu.VMEM((N_DEV, SQ, D), jnp.float32),
            pltpu.VMEM((N_DEV, SQ, D), jnp.float32),
            pltpu.VMEM((2, SKV, DH), jnp.float32),
            pltpu.VMEM((2, SKV, DH), jnp.float32),
            pltpu.VMEM((SQ, D), jnp.float32),
            pltpu.VMEM((SQ, D), jnp.float32),
            pltpu.VMEM((N_DEV - 1, SQ, D), jnp.float32),
            pltpu.SemaphoreType.DMA((N_DEV - 1,)),
            pltpu.SemaphoreType.DMA((N_DEV - 1,)),
            pltpu.SemaphoreType.DMA((N_DEV - 1,)),
            pltpu.SemaphoreType.DMA((N_DEV - 1,)),
            pltpu.SemaphoreType.DMA((2,)),
            pltpu.SemaphoreType.DMA((2,)),
        ],
        compiler_params=pltpu.CompilerParams(
            collective_id=0,
            vmem_limit_bytes=60 * 1024 * 1024,
        ),
    )(x2, Wq, Wo, K_ext, V_ext)
